# baseline (speedup 1.0000x reference)
import functools

import jax
import jax.numpy as jnp
from jax import lax
from jax.experimental import pallas as pl
from jax.experimental.pallas import tpu as pltpu
from jax.experimental.pallas import tpu_sc as plsc

NODE_NUM = 100
BATCH = 1024
FEAT = 128
SENSOR_DIM = 64
TOPK = 20
KSIZE = 3
N_EDGES = 2000

BBLK = 64


def _shift_time(h, s):
    if s == 0:
        return h
    bblk, l, c = h.shape
    z = jnp.zeros((bblk, s, c), dtype=h.dtype)
    return jnp.concatenate([z, h[:, : l - s, :]], axis=1)


def _tcn_kernel(x_ref, v1_ref, g1_ref, b1_ref, v2_ref, g2_ref, b2_ref,
                v3_ref, g3_ref, b3_ref, o_ref, w_ref):
    vrefs = (v1_ref, v2_ref, v3_ref)
    grefs = (g1_ref, g2_ref, g3_ref)
    brefs = (b1_ref, b2_ref, b3_ref)

    @pl.when(pl.program_id(0) == 0)
    def _():
        for i in range(3):
            v = vrefs[i][...]
            g = grefs[i][...]
            sumsq = jnp.sum(v * v, axis=0)
            sumsq = jnp.sum(sumsq, axis=1, keepdims=True)
            scale = jnp.transpose(g, (1, 0)) / jnp.sqrt(sumsq + 1e-12)
            for k in range(KSIZE):
                w_ref[i * KSIZE + k] = v[k] * scale

    x = x_ref[...]
    xt = jnp.transpose(x, (0, 2, 1))
    h = xt
    for i, d in enumerate((1, 2, 4)):
        acc = brefs[i][...][None]
        for k in range(KSIZE):
            w = w_ref[i * KSIZE + k].astype(jnp.bfloat16)
            hs = _shift_time(h, (KSIZE - 1 - k) * d).astype(jnp.bfloat16)
            acc = acc + lax.dot_general(
                hs, w, (((2,), (1,)), ((), ())),
                preferred_element_type=jnp.float32)
        h = jnp.maximum(acc, 0.0)
    out = jnp.maximum(h + xt, 0.0)
    o_ref[...] = jnp.transpose(out, (0, 2, 1))


NPAD = 112


def _cos_kernel(e1_ref, e2_ref, cos_ref):
    w1 = e1_ref[...]
    w2 = e2_ref[...]
    dots = lax.dot_general(w1, w2, (((1,), (1,)), ((), ())),
                           preferred_element_type=jnp.float32)
    n1 = jnp.sqrt(jnp.sum(w1 * w1, axis=1, keepdims=True))
    n2 = jnp.sqrt(jnp.sum(w2 * w2, axis=1, keepdims=True))
    n2row = jnp.transpose(n2, (1, 0))
    cos = jnp.maximum(dots, 0.0) / (n1 * n2row)
    pad = jnp.full((NODE_NUM, NPAD - NODE_NUM), -1.0, jnp.float32)
    cos_ref[...] = jnp.concatenate([cos, pad], axis=1)


NW = 32
BPW = BATCH // NW
EB = 8
NPAIR = NODE_NUM // 2


def _sc_body(cos_hbm, org_hbm, masked_hbm, idx_hbm, edges_hbm,
             org_v, ebuf, cpair, mbuf, ibuf):
    wid = lax.axis_index("s") * 2 + lax.axis_index("c")

    pltpu.sync_copy(org_hbm, org_v)
    b0 = wid * BPW

    def eb(bb, carry):
        for r in range(2):
            def inner(j, c):
                sl = pl.ds(j * 16, 16)
                base = org_v[r, sl]
                for e in range(EB):
                    ebuf[e, sl] = base + (b0 + bb * EB + e) * NODE_NUM
                return c
            lax.fori_loop(0, N_EDGES // 16, inner, 0)
            pltpu.sync_copy(ebuf, edges_hbm.at[r, pl.ds(b0 + bb * EB, EB)])
        return carry

    lax.fori_loop(0, BPW // EB, eb, 0)

    def do_pair(p):
        pltpu.sync_copy(cos_hbm.at[pl.ds(p * 2, 2)], cpair)
        iota = lax.iota(jnp.int32, 16)
        nchunk = NPAD // 16
        for i in range(2):
            chunks = [cpair[i, pl.ds(j * 16, 16)] for j in range(nchunk)]
            zi = jnp.zeros((16,), jnp.int32)
            zf = jnp.zeros((16,), jnp.float32)
            carry0 = tuple(chunks) + (zi, zi, zf, zf)

            def step(t, carry):
                ws = list(carry[:nchunk])
                acc1, acc2, vacc1, vacc2 = carry[nchunk:]
                v = ws[0]
                for j in range(1, nchunk):
                    v = jnp.maximum(v, ws[j])
                s = v[0]
                for l in range(1, 16):
                    s = jnp.maximum(s, v[l])
                mn = jnp.full((16,), 9999, jnp.int32)
                for j in range(nchunk):
                    cand = jnp.where(ws[j] == s, iota + j * 16, 9999)
                    mn = jnp.minimum(mn, cand)
                b = mn[0]
                for l in range(1, 16):
                    b = jnp.minimum(b, mn[l])
                for j in range(nchunk):
                    ws[j] = jnp.where((iota + j * 16) == b, -1.0, ws[j])
                acc1 = jnp.where(iota == t, b, acc1)
                acc2 = jnp.where(iota == t - 4, b, acc2)
                vacc1 = jnp.where(iota == t, s, vacc1)
                vacc2 = jnp.where(iota == t - 4, s, vacc2)
                return tuple(ws) + (acc1, acc2, vacc1, vacc2)

            res = lax.fori_loop(0, TOPK, step, carry0)
            ws = res[:nchunk]
            acc1, acc2, vacc1, vacc2 = res[nchunk:]
            for j in range(nchunk):
                mbuf[i, pl.ds(j * 16, 16)] = jnp.where(
                    ws[j] < 0.0, chunks[j], 0.0)
            ibuf[i, pl.ds(0, 16)] = acc1
            ibuf[i, pl.ds(4, 16)] = acc2
        pltpu.sync_copy(mbuf, masked_hbm.at[pl.ds(p * 2, 2)])
        pltpu.sync_copy(ibuf, idx_hbm.at[pl.ds(p * 2, 2)])

    do_pair(wid)

    @pl.when(wid < NPAIR - NW)
    def _():
        do_pair(wid + NW)


def kernel(data, org_edge_index, v1, g1, b1, v2, g2, b2, v3, g3, b3,
           emb1, emb2):
    vt1 = jnp.transpose(v1, (2, 0, 1))
    vt2 = jnp.transpose(v2, (2, 0, 1))
    vt3 = jnp.transpose(v3, (2, 0, 1))
    g1r, b1r = g1.reshape(1, NODE_NUM), b1.reshape(1, NODE_NUM)
    g2r, b2r = g2.reshape(1, NODE_NUM), b2.reshape(1, NODE_NUM)
    g3r, b3r = g3.reshape(1, NODE_NUM), b3.reshape(1, NODE_NUM)

    full = lambda shape: pl.BlockSpec(shape, lambda i: (0,) * len(shape))
    x3 = pl.pallas_call(
        _tcn_kernel,
        grid=(BATCH // BBLK,),
        in_specs=[
            pl.BlockSpec((BBLK, NODE_NUM, FEAT), lambda i: (i, 0, 0)),
            full((KSIZE, NODE_NUM, NODE_NUM)), full((1, NODE_NUM)), full((1, NODE_NUM)),
            full((KSIZE, NODE_NUM, NODE_NUM)), full((1, NODE_NUM)), full((1, NODE_NUM)),
            full((KSIZE, NODE_NUM, NODE_NUM)), full((1, NODE_NUM)), full((1, NODE_NUM)),
        ],
        out_specs=pl.BlockSpec((BBLK, NODE_NUM, FEAT), lambda i: (i, 0, 0)),
        out_shape=jax.ShapeDtypeStruct((BATCH, NODE_NUM, FEAT), jnp.float32),
        scratch_shapes=[pltpu.VMEM((9, NODE_NUM, NODE_NUM), jnp.float32)],
        compiler_params=pltpu.CompilerParams(
            dimension_semantics=("arbitrary",)),
    )(data, vt1, g1r, b1r, vt2, g2r, b2r, vt3, g3r, b3r)
    x = x3.reshape(-1, FEAT)

    cos = pl.pallas_call(
        _cos_kernel,
        in_specs=[
            pl.BlockSpec((NODE_NUM, SENSOR_DIM), lambda: (0, 0)),
            pl.BlockSpec((NODE_NUM, SENSOR_DIM), lambda: (0, 0)),
        ],
        out_specs=pl.BlockSpec((NODE_NUM, NPAD), lambda: (0, 0)),
        out_shape=jax.ShapeDtypeStruct((NODE_NUM, NPAD), jnp.float32),
    )(emb1, emb2)

    mesh = plsc.VectorSubcoreMesh(core_axis_name="c", subcore_axis_name="s")
    sc = pl.kernel(
        _sc_body,
        mesh=mesh,
        out_type=[
            jax.ShapeDtypeStruct((NODE_NUM, NPAD), jnp.float32),
            jax.ShapeDtypeStruct((NODE_NUM, TOPK), jnp.int32),
            jax.ShapeDtypeStruct((2, BATCH, N_EDGES), jnp.int32),
        ],
        scratch_types=[
            pltpu.VMEM((2, N_EDGES), jnp.int32),
            pltpu.VMEM((EB, N_EDGES), jnp.int32),
            pltpu.VMEM((2, NPAD), jnp.float32),
            pltpu.VMEM((2, NPAD), jnp.float32),
            pltpu.VMEM((2, TOPK), jnp.int32),
        ],
    )
    masked_pad, idx, edges = sc(cos, org_edge_index)
    masked = masked_pad[:, :NODE_NUM]
    batch_edge_index = edges.reshape(2, BATCH * N_EDGES)

    return x, masked, idx, batch_edge_index


# speedup vs baseline: 1.5192x; 1.5192x over previous
import functools

import jax
import jax.numpy as jnp
from jax import lax
from jax.experimental import pallas as pl
from jax.experimental.pallas import tpu as pltpu
from jax.experimental.pallas import tpu_sc as plsc

NODE_NUM = 100
BATCH = 1024
FEAT = 128
SENSOR_DIM = 64
TOPK = 20
KSIZE = 3
N_EDGES = 2000

BBLK = 64


def _shift_time(h, s):
    if s == 0:
        return h
    bblk, l, c = h.shape
    z = jnp.zeros((bblk, s, c), dtype=h.dtype)
    return jnp.concatenate([z, h[:, : l - s, :]], axis=1)


def _tcn_kernel(x_ref, v1_ref, g1_ref, b1_ref, v2_ref, g2_ref, b2_ref,
                v3_ref, g3_ref, b3_ref, o_ref, w_ref):
    vrefs = (v1_ref, v2_ref, v3_ref)
    grefs = (g1_ref, g2_ref, g3_ref)
    brefs = (b1_ref, b2_ref, b3_ref)

    @pl.when(pl.program_id(0) == 0)
    def _():
        for i in range(3):
            v = vrefs[i][...]
            g = grefs[i][...]
            sumsq = jnp.sum(v * v, axis=0)
            sumsq = jnp.sum(sumsq, axis=1, keepdims=True)
            scale = jnp.transpose(g, (1, 0)) / jnp.sqrt(sumsq + 1e-12)
            for k in range(KSIZE):
                w_ref[i * KSIZE + k] = v[k] * scale

    x = x_ref[...]
    xt = jnp.transpose(x, (0, 2, 1))
    h = xt
    for i, d in enumerate((1, 2, 4)):
        acc = brefs[i][...][None]
        for k in range(KSIZE):
            w = w_ref[i * KSIZE + k]
            hs = _shift_time(h, (KSIZE - 1 - k) * d)
            acc = acc + lax.dot_general(
                hs, w, (((2,), (1,)), ((), ())),
                preferred_element_type=jnp.float32)
        h = jnp.maximum(acc, 0.0)
    out = jnp.maximum(h + xt, 0.0)
    o_ref[...] = jnp.transpose(out, (0, 2, 1)).reshape(BBLK * NODE_NUM, FEAT)


NPAD = 112


def _cos_kernel(e1_ref, e2_ref, cos_ref):
    w1 = e1_ref[...]
    w2 = e2_ref[...]
    dots = lax.dot_general(w1, w2, (((1,), (1,)), ((), ())),
                           preferred_element_type=jnp.float32)
    n1 = jnp.sqrt(jnp.sum(w1 * w1, axis=1, keepdims=True))
    n2 = jnp.sqrt(jnp.sum(w2 * w2, axis=1, keepdims=True))
    n2row = jnp.transpose(n2, (1, 0))
    cos = jnp.maximum(dots, 0.0) / (n1 * n2row)
    pad = jnp.full((NODE_NUM, NPAD - NODE_NUM), -1.0, jnp.float32)
    cos_ref[...] = jnp.concatenate([cos, pad], axis=1)


NW = 32
BPW = BATCH // NW
EB = 8
NPAIR = NODE_NUM // 2


def _sc_body(cos_hbm, org_hbm, masked_hbm, idx_hbm, edges_hbm,
             org_v, ebuf, cpair, mbuf, ibuf):
    wid = lax.axis_index("s") * 2 + lax.axis_index("c")

    pltpu.sync_copy(org_hbm, org_v)
    b0 = wid * BPW

    def eb(bb, carry):
        for r in range(2):
            def inner(j, c):
                base = org_v[r, pl.ds(j * 16, 16)]
                for e in range(EB):
                    ebuf[pl.ds(e * N_EDGES + j * 16, 16)] = (
                        base + (b0 + bb * EB + e) * NODE_NUM)
                return c
            lax.fori_loop(0, N_EDGES // 16, inner, 0)
            pltpu.sync_copy(
                ebuf, edges_hbm.at[r, pl.ds((b0 + bb * EB) * N_EDGES,
                                            EB * N_EDGES)])
        return carry

    lax.fori_loop(0, BPW // EB, eb, 0)

    def do_pair(p):
        pltpu.sync_copy(cos_hbm.at[pl.ds(p * 2, 2)], cpair)
        iota = lax.iota(jnp.int32, 16)
        nchunk = NPAD // 16
        for i in range(2):
            chunks = [cpair[i, pl.ds(j * 16, 16)] for j in range(nchunk)]
            zi = jnp.zeros((16,), jnp.int32)
            zf = jnp.zeros((16,), jnp.float32)
            carry0 = tuple(chunks) + (zi, zi, zf, zf)

            def step(t, carry):
                ws = list(carry[:nchunk])
                acc1, acc2, vacc1, vacc2 = carry[nchunk:]
                v = ws[0]
                for j in range(1, nchunk):
                    v = jnp.maximum(v, ws[j])
                s = v[0]
                for l in range(1, 16):
                    s = jnp.maximum(s, v[l])
                mn = jnp.full((16,), 9999, jnp.int32)
                for j in range(nchunk):
                    cand = jnp.where(ws[j] == s, iota + j * 16, 9999)
                    mn = jnp.minimum(mn, cand)
                b = mn[0]
                for l in range(1, 16):
                    b = jnp.minimum(b, mn[l])
                for j in range(nchunk):
                    ws[j] = jnp.where((iota + j * 16) == b, -1.0, ws[j])
                acc1 = jnp.where(iota == t, b, acc1)
                acc2 = jnp.where(iota == t - 4, b, acc2)
                vacc1 = jnp.where(iota == t, s, vacc1)
                vacc2 = jnp.where(iota == t - 4, s, vacc2)
                return tuple(ws) + (acc1, acc2, vacc1, vacc2)

            res = lax.fori_loop(0, TOPK, step, carry0)
            ws = res[:nchunk]
            acc1, acc2, vacc1, vacc2 = res[nchunk:]
            for j in range(nchunk):
                mbuf[i, pl.ds(j * 16, 16)] = jnp.where(
                    ws[j] < 0.0, chunks[j], 0.0)
            ibuf[i, pl.ds(0, 16)] = acc1
            ibuf[i, pl.ds(4, 16)] = acc2
        pltpu.sync_copy(mbuf, masked_hbm.at[pl.ds(p * 2, 2)])
        pltpu.sync_copy(ibuf, idx_hbm.at[pl.ds(p * 2, 2)])

    do_pair(wid)

    @pl.when(wid < NPAIR - NW)
    def _():
        do_pair(wid + NW)


def kernel(data, org_edge_index, v1, g1, b1, v2, g2, b2, v3, g3, b3,
           emb1, emb2):
    vt1 = jnp.transpose(v1, (2, 0, 1))
    vt2 = jnp.transpose(v2, (2, 0, 1))
    vt3 = jnp.transpose(v3, (2, 0, 1))
    g1r, b1r = g1.reshape(1, NODE_NUM), b1.reshape(1, NODE_NUM)
    g2r, b2r = g2.reshape(1, NODE_NUM), b2.reshape(1, NODE_NUM)
    g3r, b3r = g3.reshape(1, NODE_NUM), b3.reshape(1, NODE_NUM)

    full = lambda shape: pl.BlockSpec(shape, lambda i: (0,) * len(shape))
    x3 = pl.pallas_call(
        _tcn_kernel,
        grid=(BATCH // BBLK,),
        in_specs=[
            pl.BlockSpec((BBLK, NODE_NUM, FEAT), lambda i: (i, 0, 0)),
            full((KSIZE, NODE_NUM, NODE_NUM)), full((1, NODE_NUM)), full((1, NODE_NUM)),
            full((KSIZE, NODE_NUM, NODE_NUM)), full((1, NODE_NUM)), full((1, NODE_NUM)),
            full((KSIZE, NODE_NUM, NODE_NUM)), full((1, NODE_NUM)), full((1, NODE_NUM)),
        ],
        out_specs=pl.BlockSpec((BBLK * NODE_NUM, FEAT), lambda i: (i, 0)),
        out_shape=jax.ShapeDtypeStruct((BATCH * NODE_NUM, FEAT), jnp.float32),
        scratch_shapes=[pltpu.VMEM((9, NODE_NUM, NODE_NUM), jnp.float32)],
        compiler_params=pltpu.CompilerParams(
            dimension_semantics=("arbitrary",)),
    )(data, vt1, g1r, b1r, vt2, g2r, b2r, vt3, g3r, b3r)
    x = x3

    cos = pl.pallas_call(
        _cos_kernel,
        in_specs=[
            pl.BlockSpec((NODE_NUM, SENSOR_DIM), lambda: (0, 0)),
            pl.BlockSpec((NODE_NUM, SENSOR_DIM), lambda: (0, 0)),
        ],
        out_specs=pl.BlockSpec((NODE_NUM, NPAD), lambda: (0, 0)),
        out_shape=jax.ShapeDtypeStruct((NODE_NUM, NPAD), jnp.float32),
    )(emb1, emb2)

    mesh = plsc.VectorSubcoreMesh(core_axis_name="c", subcore_axis_name="s")
    sc = pl.kernel(
        _sc_body,
        mesh=mesh,
        out_type=[
            jax.ShapeDtypeStruct((NODE_NUM, NPAD), jnp.float32),
            jax.ShapeDtypeStruct((NODE_NUM, TOPK), jnp.int32),
            jax.ShapeDtypeStruct((2, BATCH * N_EDGES), jnp.int32),
        ],
        scratch_types=[
            pltpu.VMEM((2, N_EDGES), jnp.int32),
            pltpu.VMEM((EB * N_EDGES,), jnp.int32),
            pltpu.VMEM((2, NPAD), jnp.float32),
            pltpu.VMEM((2, NPAD), jnp.float32),
            pltpu.VMEM((2, TOPK), jnp.int32),
        ],
    )
    masked_pad, idx, batch_edge_index = sc(cos, org_edge_index)
    masked = masked_pad[:, :NODE_NUM]

    return x, masked, idx, batch_edge_index
